# baseline (speedup 1.0000x reference)
import functools

import jax
import jax.numpy as jnp
from jax import lax
from jax.experimental import pallas as pl
from jax.experimental.pallas import tpu as pltpu
from jax.experimental.pallas import tpu_sc as plsc

D_MODEL = 128
N_ROWS = 4096 * 200
NC, NS = 2, 16
NW = NC * NS
ROWS_PER_W = N_ROWS // NW
CHUNK = 128
NCHUNK = ROWS_PER_W // CHUNK
NBUF = 4


@functools.partial(
    pl.kernel,
    out_type=jax.ShapeDtypeStruct((N_ROWS, D_MODEL), jnp.float32),
    mesh=plsc.VectorSubcoreMesh(core_axis_name="c", subcore_axis_name="s"),
    scratch_types=[
        pltpu.VMEM((NCHUNK, CHUNK), jnp.int32),
        [pltpu.VMEM((CHUNK, D_MODEL), jnp.float32) for _ in range(NBUF)],
        [pltpu.SemaphoreType.DMA for _ in range(NBUF)],
    ],
)
def _gather_rows(idx_hbm, pe_hbm, out_hbm, idx_v, rows, sems):
    wid = lax.axis_index("s") * NC + lax.axis_index("c")
    base = wid * ROWS_PER_W
    pltpu.sync_copy(idx_hbm.at[wid], idx_v)

    for b in range(NBUF):
        pltpu.async_copy(pe_hbm.at[idx_v.at[b]], rows[b], sems[b])

    def step(j0, carry):
        for b in range(NBUF):
            j = j0 * NBUF + b
            pltpu.make_async_copy(pe_hbm.at[idx_v.at[j]], rows[b], sems[b]).wait()
            pltpu.sync_copy(rows[b], out_hbm.at[pl.ds(base + j * CHUNK, CHUNK)])

            @pl.when(j + NBUF < NCHUNK)
            def _():
                pltpu.async_copy(pe_hbm.at[idx_v.at[j + NBUF]], rows[b], sems[b])

        return carry

    lax.fori_loop(0, NCHUNK // NBUF, step, 0)


def kernel(days, pe):
    idx = days.reshape(NW, NCHUNK, CHUNK)
    out = _gather_rows(idx, pe)
    return out.reshape(days.shape[0], days.shape[1], D_MODEL)


# speedup vs baseline: 3.9628x; 3.9628x over previous
import functools

import jax
import jax.numpy as jnp
from jax import lax
from jax.experimental import pallas as pl
from jax.experimental.pallas import tpu as pltpu
from jax.experimental.pallas import tpu_sc as plsc

D_MODEL = 128
N_ROWS = 4096 * 200
NC, NS = 2, 16
NW = NC * NS
ROWS_PER_W = N_ROWS // NW
CHUNK = 128
NCHUNK = ROWS_PER_W // CHUNK
NBUF = 4
MAX_ROWS = 398


@functools.partial(
    pl.kernel,
    out_type=jax.ShapeDtypeStruct((N_ROWS, D_MODEL), jnp.float32),
    mesh=plsc.VectorSubcoreMesh(core_axis_name="c", subcore_axis_name="s"),
    scratch_types=[
        pltpu.VMEM((NCHUNK, CHUNK), jnp.int32),
        pltpu.VMEM_SHARED((MAX_ROWS, D_MODEL), jnp.float32),
        [pltpu.VMEM((CHUNK, D_MODEL), jnp.float32) for _ in range(NBUF)],
        [pltpu.SemaphoreType.DMA for _ in range(NBUF)],
    ],
)
def _gather_rows(idx_hbm, pe_hbm, out_hbm, idx_v, table_sh, rows, sems):
    wid = lax.axis_index("s") * NC + lax.axis_index("c")
    base = wid * ROWS_PER_W

    @pl.when(lax.axis_index("s") == 0)
    def _():
        pltpu.sync_copy(pe_hbm, table_sh)

    pltpu.sync_copy(idx_hbm.at[wid], idx_v)
    plsc.subcore_barrier()

    for b in range(NBUF):
        pltpu.async_copy(table_sh.at[idx_v.at[b]], rows[b], sems[b])

    def step(j0, carry):
        for b in range(NBUF):
            j = j0 * NBUF + b
            pltpu.make_async_copy(table_sh.at[idx_v.at[j]], rows[b], sems[b]).wait()
            pltpu.sync_copy(rows[b], out_hbm.at[pl.ds(base + j * CHUNK, CHUNK)])

            @pl.when(j + NBUF < NCHUNK)
            def _():
                pltpu.async_copy(table_sh.at[idx_v.at[j + NBUF]], rows[b], sems[b])

        return carry

    lax.fori_loop(0, NCHUNK // NBUF, step, 0)


def kernel(days, pe):
    idx = days.reshape(NW, NCHUNK, CHUNK)
    out = _gather_rows(idx, pe)
    return out.reshape(days.shape[0], days.shape[1], D_MODEL)
